# baseline (speedup 1.0000x reference)
import functools

import jax
import jax.numpy as jnp
from jax import lax
from jax.experimental import pallas as pl
from jax.experimental.pallas import tpu as pltpu
from jax.experimental.pallas import tpu_sc as plsc

N = 10000
E = 320000
NPAD = 10016
NW = 32
EPW = E // NW
CAP = 784
SLICE_W = 16 * CAP

_MESH = plsc.VectorSubcoreMesh(core_axis_name="c", subcore_axis_name="s")
_SC_PARAMS = pltpu.CompilerParams(needs_layout_passes=False)


def _wid():
    return lax.axis_index("s") * 2 + lax.axis_index("c")


def _lane_gather(vec, idx):
    return lax.gather(
        vec, idx[:, None],
        lax.GatherDimensionNumbers((), (0,), (0,)), (1,),
        mode=lax.GatherScatterMode.PROMISE_IN_BOUNDS)


@functools.partial(
    pl.kernel,
    out_type=[
        jax.ShapeDtypeStruct((NW * SLICE_W,), jnp.int32),
        jax.ShapeDtypeStruct((NW * 16,), jnp.int32),
        jax.ShapeDtypeStruct((NW * NPAD,), jnp.float32),
    ],
    mesh=_MESH,
    compiler_params=_SC_PARAMS,
    scratch_types=[
        pltpu.VMEM((EPW,), jnp.int32),
        pltpu.VMEM((EPW,), jnp.int32),
        pltpu.VMEM((NPAD,), jnp.float32),
        pltpu.VMEM((16 * (CAP + 8),), jnp.int32),
        pltpu.VMEM((16,), jnp.int32),
        pltpu.VMEM((16,), jnp.int32),
    ],
)
def _sc_prep(src_hbm, dst_hbm, eb_hbm, meta_hbm, hist_hbm,
             sbuf, dbuf, hist, obuf, cntv, kbuf):
    wid = _wid()
    base = wid * EPW
    pltpu.sync_copy(src_hbm.at[pl.ds(base, EPW)], sbuf)
    pltpu.sync_copy(dst_hbm.at[pl.ds(base, EPW)], dbuf)

    @pl.loop(0, NPAD, step=16)
    def _(i):
        hist[pl.ds(i, 16)] = jnp.zeros((16,), jnp.float32)

    cntv[...] = jnp.zeros((16,), jnp.int32)
    iota = lax.iota(jnp.int32, 16)
    ones_f = jnp.ones((16,), jnp.float32)
    ones_i = jnp.ones((16,), jnp.int32)

    @pl.loop(0, EPW, step=16)
    def _(i):
        s = sbuf[pl.ds(i, 16)]
        d = dbuf[pl.ds(i, 16)]
        dadj = jnp.where(s != d, d, N)
        plsc.addupdate_scatter(hist, [dadj], ones_f)
        v = (s << 14) | dadj
        r = dadj & 15
        rs, vs = plsc.sort_key_val(r, v)
        prev = _lane_gather(rs, jnp.maximum(iota - 1, 0))
        seg = jnp.where(rs != prev, iota, 0)
        rank = iota - plsc.cummax(seg)
        c = plsc.load_gather(cntv, [rs])
        pos = (jnp.minimum(c + rank, CAP + 6) << 4) | rs
        plsc.store_scatter(obuf, [pos], vs)
        plsc.addupdate_scatter(cntv, [rs], ones_i)

    cfin = cntv[...]
    k = jnp.minimum(jnp.max(cfin), CAP)
    dump = jnp.int32(N) + iota

    @pl.loop(jnp.min(cfin), k)
    def _(j):
        plsc.store_scatter(obuf, [(j << 4) | iota], dump, mask=cfin <= j)

    kbuf[...] = jnp.full((16,), k, jnp.int32)
    pltpu.sync_copy(obuf.at[pl.ds(0, SLICE_W)],
                    eb_hbm.at[pl.ds(wid * SLICE_W, SLICE_W)])
    pltpu.sync_copy(kbuf, meta_hbm.at[pl.ds(wid * 16, 16)])
    pltpu.sync_copy(hist, hist_hbm.at[pl.ds(wid * NPAD, NPAD)])


def _make_sc_conv(nch, unroll):
    npair = nch // 2
    c_total = NW * nch
    half = c_total // 2

    @functools.partial(
        pl.kernel,
        out_type=jax.ShapeDtypeStruct((c_total * N,), jnp.float32),
        mesh=_MESH,
        compiler_params=_SC_PARAMS,
        scratch_types=[
            pltpu.VMEM((npair * NPAD,), jnp.int32),
            pltpu.VMEM((nch * NPAD,), jnp.float32),
            pltpu.VMEM((2 * SLICE_W,), jnp.int32),
            pltpu.VMEM((NW * 16,), jnp.int32),
            pltpu.SemaphoreType.DMA,
        ],
    )
    def conv(eb_hbm, meta_hbm, hp_hbm, agg_hbm, htab, acc, ebuf, kv, sem):
        wid = _wid()
        p0 = wid * npair
        pltpu.sync_copy(meta_hbm, kv)
        for j in range(npair):
            pltpu.sync_copy(hp_hbm.at[pl.ds((p0 + j) * N, N)],
                            htab.at[pl.ds(j * NPAD, N)])

        @pl.loop(0, nch * NPAD, step=16)
        def _(i):
            acc[pl.ds(i, 16)] = jnp.zeros((16,), jnp.float32)

        def start(si, off):
            pltpu.async_copy(eb_hbm.at[pl.ds(si * SLICE_W, SLICE_W)],
                             ebuf.at[pl.ds(off, SLICE_W)], sem)

        def wait(si, off):
            pltpu.make_async_copy(eb_hbm.at[pl.ds(si * SLICE_W, SLICE_W)],
                                  ebuf.at[pl.ds(off, SLICE_W)], sem).wait()

        start(0, 0)

        @pl.loop(0, NW)
        def _(si):
            off = lax.rem(si, 2) * SLICE_W

            @pl.when(si + 1 < NW)
            def _():
                start(si + 1, SLICE_W - off)

            wait(si, off)
            k16 = kv[pl.ds(si * 16, 16)][0] << 4

            @plsc.parallel_loop(0, k16, step=16, unroll=unroll)
            def _(i):
                p = ebuf[pl.ds(off + i, 16)]
                s = p >> 14
                d = p & 16383
                for j in range(npair):
                    g = plsc.load_gather(htab, [s + (j * NPAD)])
                    glo = plsc.bitcast(g << 16, jnp.float32)
                    ghi = plsc.bitcast(g & jnp.int32(-65536), jnp.float32)
                    plsc.addupdate_scatter(acc, [d + (j * NPAD)], glo)
                    plsc.addupdate_scatter(acc, [d + ((npair + j) * NPAD)], ghi)

        for j in range(npair):
            pltpu.sync_copy(acc.at[pl.ds(j * NPAD, N)],
                            agg_hbm.at[pl.ds((p0 + j) * N, N)])
            pltpu.sync_copy(acc.at[pl.ds((npair + j) * NPAD, N)],
                            agg_hbm.at[pl.ds((p0 + j + half) * N, N)])

    return conv


_sc_conv64 = _make_sc_conv(2, 16)
_sc_conv128 = _make_sc_conv(4, 8)


def _pack_pairs(hs):
    c = hs.shape[0]
    lo = lax.bitcast_convert_type(hs[: c // 2].astype(jnp.bfloat16),
                                  jnp.uint16).astype(jnp.uint32)
    hi = lax.bitcast_convert_type(hs[c // 2:].astype(jnp.bfloat16),
                                  jnp.uint16).astype(jnp.uint32)
    return lax.bitcast_convert_type(lo | (hi << 16), jnp.int32)


def _tc_first(W, x, hist):
    def body(w_ref, x_ref, hist_ref, o_ref, hp_ref, dinv_ref, deginv_ref):
        h = lax.dot_general(
            w_ref[...], x_ref[...], (((1,), (1,)), ((), ())),
            preferred_element_type=jnp.float32,
            precision=lax.Precision.HIGHEST)
        deg = jnp.sum(hist_ref[...], axis=0, keepdims=True)[:, :N] + 1.0
        dinv = lax.rsqrt(deg)
        o_ref[...] = h
        hp_ref[...] = _pack_pairs(dinv * h)
        dinv_ref[...] = dinv
        deginv_ref[...] = 1.0 / deg

    return pl.pallas_call(
        body,
        out_shape=[
            jax.ShapeDtypeStruct((W.shape[0], x.shape[0]), jnp.float32),
            jax.ShapeDtypeStruct((W.shape[0] // 2, x.shape[0]), jnp.int32),
            jax.ShapeDtypeStruct((1, N), jnp.float32),
            jax.ShapeDtypeStruct((1, N), jnp.float32),
        ],
    )(W, x, hist)


def _tc_stage(aggT, hT, dinv, deginv, b, Wn):
    def body(a_ref, h_ref, di_ref, dg_ref, b_ref, w_ref, o_ref, hp_ref):
        t = di_ref[...] * a_ref[...] + dg_ref[...] * h_ref[...] + b_ref[...]
        mean = jnp.mean(t, axis=1, keepdims=True)
        cen = t - mean
        var = jnp.mean(cen * cen, axis=1, keepdims=True)
        z = jnp.maximum(cen * lax.rsqrt(var + 1e-5), 0.0)
        h = jnp.dot(w_ref[...], z,
                    preferred_element_type=jnp.float32,
                    precision=lax.Precision.HIGHEST)
        o_ref[...] = h
        hp_ref[...] = _pack_pairs(di_ref[...] * h)

    return pl.pallas_call(
        body,
        out_shape=[
            jax.ShapeDtypeStruct((Wn.shape[0], aggT.shape[1]), jnp.float32),
            jax.ShapeDtypeStruct((Wn.shape[0] // 2, aggT.shape[1]), jnp.int32),
        ],
    )(aggT, hT, dinv, deginv, b, Wn)


def _tc_final(aggT, hT, dinv, deginv, b):
    def body(a_ref, h_ref, di_ref, dg_ref, b_ref, o_ref):
        t = di_ref[...] * a_ref[...] + dg_ref[...] * h_ref[...] + b_ref[...]
        m = jnp.max(t, axis=0, keepdims=True)
        lse = jnp.log(jnp.sum(jnp.exp(t - m), axis=0, keepdims=True)) + m
        o_ref[...] = t - lse

    return pl.pallas_call(
        body,
        out_shape=jax.ShapeDtypeStruct(aggT.shape, jnp.float32),
    )(aggT, hT, dinv, deginv, b)


def kernel(x, edge_index, W0, b0, W1, b1, W2, b2):
    src = edge_index[0].astype(jnp.int32)
    dst = edge_index[1].astype(jnp.int32)

    ebkt, meta, hist = _sc_prep(src, dst)
    h0T, hp0, dinv, deginv = _tc_first(W0, x, hist.reshape(NW, NPAD))

    agg0 = _sc_conv64(ebkt, meta, hp0.reshape(-1)).reshape(64, N)
    h1T, hp1 = _tc_stage(agg0, h0T, dinv, deginv, b0.reshape(-1, 1), W1)
    agg1 = _sc_conv64(ebkt, meta, hp1.reshape(-1)).reshape(64, N)
    h2T, hp2 = _tc_stage(agg1, h1T, dinv, deginv, b1.reshape(-1, 1), W2)
    agg2 = _sc_conv128(ebkt, meta, hp2.reshape(-1)).reshape(128, N)
    outT = _tc_final(agg2, h2T, dinv, deginv, b2.reshape(-1, 1))
    return outT.T


# speedup vs baseline: 1.0267x; 1.0267x over previous
import functools

import jax
import jax.numpy as jnp
from jax import lax
from jax.experimental import pallas as pl
from jax.experimental.pallas import tpu as pltpu
from jax.experimental.pallas import tpu_sc as plsc

N = 10000
E = 320000
NPAD = 10016
NW = 32
EPW = E // NW
CAP = 784
SLICE_W = 16 * CAP

_MESH = plsc.VectorSubcoreMesh(core_axis_name="c", subcore_axis_name="s")
_SC_PARAMS = pltpu.CompilerParams(needs_layout_passes=False)


def _wid():
    return lax.axis_index("s") * 2 + lax.axis_index("c")


def _lane_gather(vec, idx):
    return lax.gather(
        vec, idx[:, None],
        lax.GatherDimensionNumbers((), (0,), (0,)), (1,),
        mode=lax.GatherScatterMode.PROMISE_IN_BOUNDS)


@functools.partial(
    pl.kernel,
    out_type=[
        jax.ShapeDtypeStruct((NW * SLICE_W,), jnp.int32),
        jax.ShapeDtypeStruct((NW * 16,), jnp.int32),
        jax.ShapeDtypeStruct((NW * NPAD,), jnp.float32),
    ],
    mesh=_MESH,
    compiler_params=_SC_PARAMS,
    scratch_types=[
        pltpu.VMEM((EPW,), jnp.int32),
        pltpu.VMEM((EPW,), jnp.int32),
        pltpu.VMEM((NPAD,), jnp.float32),
        pltpu.VMEM((16 * (CAP + 8),), jnp.int32),
        pltpu.VMEM((16,), jnp.int32),
        pltpu.VMEM((16,), jnp.int32),
    ],
)
def _sc_prep(src_hbm, dst_hbm, eb_hbm, meta_hbm, hist_hbm,
             sbuf, dbuf, hist, obuf, cntv, kbuf):
    wid = _wid()
    base = wid * EPW
    pltpu.sync_copy(src_hbm.at[pl.ds(base, EPW)], sbuf)
    pltpu.sync_copy(dst_hbm.at[pl.ds(base, EPW)], dbuf)

    @pl.loop(0, NPAD, step=16)
    def _(i):
        hist[pl.ds(i, 16)] = jnp.zeros((16,), jnp.float32)

    cntv[...] = jnp.zeros((16,), jnp.int32)
    iota = lax.iota(jnp.int32, 16)
    ones_f = jnp.ones((16,), jnp.float32)
    ones_i = jnp.ones((16,), jnp.int32)

    @pl.loop(0, EPW, step=16)
    def _(i):
        s = sbuf[pl.ds(i, 16)]
        d = dbuf[pl.ds(i, 16)]
        dadj = jnp.where(s != d, d, N)
        plsc.addupdate_scatter(hist, [dadj], ones_f)
        v = (s << 14) | dadj
        r = dadj & 15
        rs, vs = plsc.sort_key_val(r, v)
        prev = _lane_gather(rs, jnp.maximum(iota - 1, 0))
        seg = jnp.where(rs != prev, iota, 0)
        rank = iota - plsc.cummax(seg)
        c = plsc.load_gather(cntv, [rs])
        pos = (jnp.minimum(c + rank, CAP + 6) << 4) | rs
        plsc.store_scatter(obuf, [pos], vs)
        plsc.addupdate_scatter(cntv, [rs], ones_i)

    cfin = cntv[...]
    k = jnp.minimum(jnp.max(cfin), CAP)
    dump = jnp.int32(N) + iota

    @pl.loop(jnp.min(cfin), k)
    def _(j):
        plsc.store_scatter(obuf, [(j << 4) | iota], dump, mask=cfin <= j)

    kbuf[...] = jnp.full((16,), k, jnp.int32)
    pltpu.sync_copy(obuf.at[pl.ds(0, SLICE_W)],
                    eb_hbm.at[pl.ds(wid * SLICE_W, SLICE_W)])
    pltpu.sync_copy(kbuf, meta_hbm.at[pl.ds(wid * 16, 16)])
    pltpu.sync_copy(hist, hist_hbm.at[pl.ds(wid * NPAD, NPAD)])


def _make_sc_conv(nch, unroll):
    npair = nch // 2
    c_total = NW * nch
    half = c_total // 2

    @functools.partial(
        pl.kernel,
        out_type=jax.ShapeDtypeStruct((c_total * N,), jnp.float32),
        mesh=_MESH,
        compiler_params=_SC_PARAMS,
        scratch_types=[
            pltpu.VMEM((npair * NPAD,), jnp.int32),
            pltpu.VMEM((nch * NPAD,), jnp.float32),
            pltpu.VMEM((2 * SLICE_W,), jnp.int32),
            pltpu.VMEM((NW * 16,), jnp.int32),
            pltpu.SemaphoreType.DMA,
        ],
    )
    def conv(eb_hbm, meta_hbm, hp_hbm, agg_hbm, htab, acc, ebuf, kv, sem):
        wid = _wid()
        p0 = wid * npair
        pltpu.sync_copy(meta_hbm, kv)
        for j in range(npair):
            pltpu.sync_copy(hp_hbm.at[pl.ds((p0 + j) * N, N)],
                            htab.at[pl.ds(j * NPAD, N)])

        @pl.loop(0, nch * NPAD, step=16)
        def _(i):
            acc[pl.ds(i, 16)] = jnp.zeros((16,), jnp.float32)

        def start(si, off):
            pltpu.async_copy(eb_hbm.at[pl.ds(si * SLICE_W, SLICE_W)],
                             ebuf.at[pl.ds(off, SLICE_W)], sem)

        def wait(si, off):
            pltpu.make_async_copy(eb_hbm.at[pl.ds(si * SLICE_W, SLICE_W)],
                                  ebuf.at[pl.ds(off, SLICE_W)], sem).wait()

        start(0, 0)

        @pl.loop(0, NW)
        def _(si):
            off = lax.rem(si, 2) * SLICE_W

            @pl.when(si + 1 < NW)
            def _():
                start(si + 1, SLICE_W - off)

            wait(si, off)
            k16 = kv[pl.ds(si * 16, 16)][0] << 4

            @plsc.parallel_loop(0, k16, step=16, unroll=unroll)
            def _(i):
                p = ebuf[pl.ds(off + i, 16)]
                s = p >> 14
                d = p & 16383
                for j in range(npair):
                    g = plsc.load_gather(htab, [s + (j * NPAD)])
                    glo = plsc.bitcast(g << 16, jnp.float32)
                    ghi = plsc.bitcast(g & jnp.int32(-65536), jnp.float32)
                    plsc.addupdate_scatter(acc, [d + (j * NPAD)], glo)
                    plsc.addupdate_scatter(acc, [d + ((npair + j) * NPAD)], ghi)

        for j in range(npair):
            pltpu.sync_copy(acc.at[pl.ds(j * NPAD, N)],
                            agg_hbm.at[pl.ds((p0 + j) * N, N)])
            pltpu.sync_copy(acc.at[pl.ds((npair + j) * NPAD, N)],
                            agg_hbm.at[pl.ds((p0 + j + half) * N, N)])

    return conv


_sc_conv64 = _make_sc_conv(2, 8)
_sc_conv128 = _make_sc_conv(4, 4)


def _pack_pairs(hs):
    c = hs.shape[0]
    lo = lax.bitcast_convert_type(hs[: c // 2].astype(jnp.bfloat16),
                                  jnp.uint16).astype(jnp.uint32)
    hi = lax.bitcast_convert_type(hs[c // 2:].astype(jnp.bfloat16),
                                  jnp.uint16).astype(jnp.uint32)
    return lax.bitcast_convert_type(lo | (hi << 16), jnp.int32)


def _tc_first(W, x, hist):
    def body(w_ref, x_ref, hist_ref, o_ref, hp_ref, dinv_ref, deginv_ref):
        h = lax.dot_general(
            w_ref[...], x_ref[...], (((1,), (1,)), ((), ())),
            preferred_element_type=jnp.float32)
        deg = jnp.sum(hist_ref[...], axis=0, keepdims=True)[:, :N] + 1.0
        dinv = lax.rsqrt(deg)
        o_ref[...] = h
        hp_ref[...] = _pack_pairs(dinv * h)
        dinv_ref[...] = dinv
        deginv_ref[...] = 1.0 / deg

    return pl.pallas_call(
        body,
        out_shape=[
            jax.ShapeDtypeStruct((W.shape[0], x.shape[0]), jnp.float32),
            jax.ShapeDtypeStruct((W.shape[0] // 2, x.shape[0]), jnp.int32),
            jax.ShapeDtypeStruct((1, N), jnp.float32),
            jax.ShapeDtypeStruct((1, N), jnp.float32),
        ],
    )(W, x, hist)


def _tc_stage(aggT, hT, dinv, deginv, b, Wn):
    def body(a_ref, h_ref, di_ref, dg_ref, b_ref, w_ref, o_ref, hp_ref):
        t = di_ref[...] * a_ref[...] + dg_ref[...] * h_ref[...] + b_ref[...]
        mean = jnp.mean(t, axis=1, keepdims=True)
        cen = t - mean
        var = jnp.mean(cen * cen, axis=1, keepdims=True)
        z = jnp.maximum(cen * lax.rsqrt(var + 1e-5), 0.0)
        h = jnp.dot(w_ref[...], z,
                    preferred_element_type=jnp.float32)
        o_ref[...] = h
        hp_ref[...] = _pack_pairs(di_ref[...] * h)

    return pl.pallas_call(
        body,
        out_shape=[
            jax.ShapeDtypeStruct((Wn.shape[0], aggT.shape[1]), jnp.float32),
            jax.ShapeDtypeStruct((Wn.shape[0] // 2, aggT.shape[1]), jnp.int32),
        ],
    )(aggT, hT, dinv, deginv, b, Wn)


def _tc_final(aggT, hT, dinv, deginv, b):
    def body(a_ref, h_ref, di_ref, dg_ref, b_ref, o_ref):
        t = di_ref[...] * a_ref[...] + dg_ref[...] * h_ref[...] + b_ref[...]
        m = jnp.max(t, axis=0, keepdims=True)
        lse = jnp.log(jnp.sum(jnp.exp(t - m), axis=0, keepdims=True)) + m
        o_ref[...] = t - lse

    return pl.pallas_call(
        body,
        out_shape=jax.ShapeDtypeStruct(aggT.shape, jnp.float32),
    )(aggT, hT, dinv, deginv, b)


def kernel(x, edge_index, W0, b0, W1, b1, W2, b2):
    src = edge_index[0].astype(jnp.int32)
    dst = edge_index[1].astype(jnp.int32)

    ebkt, meta, hist = _sc_prep(src, dst)
    h0T, hp0, dinv, deginv = _tc_first(W0, x, hist.reshape(NW, NPAD))

    agg0 = _sc_conv64(ebkt, meta, hp0.reshape(-1)).reshape(64, N)
    h1T, hp1 = _tc_stage(agg0, h0T, dinv, deginv, b0.reshape(-1, 1), W1)
    agg1 = _sc_conv64(ebkt, meta, hp1.reshape(-1)).reshape(64, N)
    h2T, hp2 = _tc_stage(agg1, h1T, dinv, deginv, b1.reshape(-1, 1), W2)
    agg2 = _sc_conv128(ebkt, meta, hp2.reshape(-1)).reshape(128, N)
    outT = _tc_final(agg2, h2T, dinv, deginv, b2.reshape(-1, 1))
    return outT.T
